# baseline (speedup 1.0000x reference)
import functools

import jax
import jax.numpy as jnp
from jax import lax
from jax.experimental import pallas as pl
from jax.experimental.pallas import tpu as pltpu
from jax.experimental.pallas import tpu_sc as plsc

N = 50000
R = 4
E = 160000
D = 128

NC = 2
NS = 16
L = 16

NP = 51200
EPT = E // NS
EPT_PAD = 10112
NKB = EPT_PAD // 128
NCHUNK = 20
CH = NP // NCHUNK
AGG_ROWS = CH * R
ROWS_PT = AGG_ROWS // NS

_MESH = dict(core_axis_name="c", subcore_axis_name="s",
             num_cores=NC, num_subcores=NS)


def _mof(x):
  return pl.multiple_of(x, 8)


def _rsqrt16(x):
  i = lax.bitcast_convert_type(x, jnp.int32)
  i = jnp.int32(0x5F3759DF) - lax.shift_right_logical(i, 1)
  y = lax.bitcast_convert_type(i, jnp.float32)
  for _ in range(3):
    y = y * (1.5 - 0.5 * x * y * y)
  return y


def _degnorm_body(ei2f, norms, deg, zbuf, nbuf, idxs, idxb, ones, onest):
  c = lax.axis_index("c")
  s = lax.axis_index("s")
  wpt = 4 * NP // NS

  def fz(i, _):
    zbuf[pl.ds(i * L, L)] = jnp.zeros((L,), jnp.float32)
    return 0
  lax.fori_loop(0, wpt // L, fz, 0)
  for j in range(128 // L):
    ones[pl.ds(j * L, L)] = jnp.ones((L,), jnp.float32)
    onest[pl.ds(j * L, L)] = jnp.full(
        (L,), 1.0 if j == 0 else 0.0, jnp.float32)
  pltpu.sync_copy(zbuf, deg.at[pl.ds(s * wpt, wpt)])
  plsc.subcore_barrier()

  for r2l in range(4):
    r2 = 4 * c + r2l
    pltpu.sync_copy(ei2f.at[pl.ds(_mof(r2 * E + s * EPT), EPT)],
                    idxs.at[pl.ds(0, EPT)])

    def mkidx(i, _):
      v = idxs[pl.ds(i * L, L)]
      v = jnp.clip(v, 0, NP - 1) + r2l * NP
      row = i // 8
      col = (i % 8) * L
      idxb[row, pl.ds(col, L)] = v
      return 0
    lax.fori_loop(0, EPT_PAD // L, mkidx, 0)

    def sca(kb, _):
      pltpu.sync_copy(ones, deg.at[idxb.at[kb]], add=True)
      return 0
    lax.fori_loop(0, NKB - 1, sca, 0)
    pltpu.sync_copy(onest, deg.at[idxb.at[NKB - 1]], add=True)
  plsc.subcore_barrier()

  off = s * wpt
  pltpu.sync_copy(deg.at[pl.ds(off, wpt)], nbuf)

  def nrm(i, _):
    x = jnp.maximum(nbuf[pl.ds(i * L, L)], 1.0)
    nbuf[pl.ds(i * L, L)] = _rsqrt16(x)
    return 0
  lax.fori_loop(0, wpt // L, nrm, 0)
  dsto = (4 * c + s // 4) * NP + (s % 4) * wpt
  pltpu.sync_copy(nbuf, norms.at[pl.ds(_mof(dsto), wpt)])


def _sc_degnorm(ei2f):
  f = pl.kernel(
      _degnorm_body,
      out_type=jax.ShapeDtypeStruct((2 * R * NP,), jnp.float32),
      mesh=plsc.VectorSubcoreMesh(**_MESH),
      compiler_params=pltpu.CompilerParams(needs_layout_passes=False),
      scratch_types=[
          pltpu.VMEM_SHARED((4 * NP,), jnp.float32),
          pltpu.VMEM((4 * NP // NS,), jnp.float32),
          pltpu.VMEM((4 * NP // NS,), jnp.float32),
          pltpu.VMEM((EPT_PAD,), jnp.int32),
          pltpu.VMEM((NKB, 128), jnp.int32),
          pltpu.VMEM((128,), jnp.float32),
          pltpu.VMEM((128,), jnp.float32),
      ],
  )
  return f(ei2f)


EPW = E // (NC * NS)
EPW_PAD = EPW + 16


def _coeff_body(ei2f, ewf, norms, cout, nsrc, ndst, sbuf, dbuf, wbuf, cbuf):
  c = lax.axis_index("c")
  s = lax.axis_index("s")
  wid = s * NC + c
  base = wid * EPW
  for r in range(R):
    pltpu.sync_copy(norms.at[pl.ds(2 * r * NP, NP)], nsrc)
    pltpu.sync_copy(norms.at[pl.ds((2 * r + 1) * NP, NP)], ndst)
    pltpu.sync_copy(ei2f.at[pl.ds(_mof(2 * r * E + base), EPW)],
                    sbuf.at[pl.ds(0, EPW)])
    pltpu.sync_copy(ei2f.at[pl.ds(_mof((2 * r + 1) * E + base), EPW)],
                    dbuf.at[pl.ds(0, EPW)])
    pltpu.sync_copy(ewf.at[pl.ds(_mof(r * E + base), EPW)],
                    wbuf.at[pl.ds(0, EPW)])

    def one(i, _):
      sv = jnp.clip(sbuf[pl.ds(i * L, L)], 0, NP - 1)
      dv = jnp.clip(dbuf[pl.ds(i * L, L)], 0, NP - 1)
      ns = plsc.load_gather(nsrc, [sv])
      nd = plsc.load_gather(ndst, [dv])
      w = wbuf[pl.ds(i * L, L)]
      cbuf[pl.ds(i * L, L)] = 0.25 * w * ns * nd
      return 0
    lax.fori_loop(0, (EPW + L - 1) // L, one, 0)
    pltpu.sync_copy(cbuf.at[pl.ds(0, EPW)],
                    cout.at[pl.ds(_mof(r * E + base), EPW)])


def _sc_coeff(ei2f, ewf, norms):
  f = pl.kernel(
      _coeff_body,
      out_type=jax.ShapeDtypeStruct((R * E,), jnp.float32),
      mesh=plsc.VectorSubcoreMesh(**_MESH),
      compiler_params=pltpu.CompilerParams(needs_layout_passes=False),
      scratch_types=[
          pltpu.VMEM((NP,), jnp.float32),
          pltpu.VMEM((NP,), jnp.float32),
          pltpu.VMEM((EPW_PAD,), jnp.int32),
          pltpu.VMEM((EPW_PAD,), jnp.int32),
          pltpu.VMEM((EPW_PAD,), jnp.float32),
          pltpu.VMEM((EPW_PAD,), jnp.float32),
      ],
  )
  return f(ei2f, ewf, norms)


KB = 160
ZR = 32
ESUB = 2000


def _agg_flush(srcb, gidb, cb, gid2d, rows, sem, x_hbm, agg):
  for j in range(128 // L):
    gid2d[0, pl.ds(j * L, L)] = gidb[pl.ds(j * L, L)]
  pltpu.async_copy(x_hbm.at[srcb.at[pl.ds(0, 128)]], rows, sem).wait()

  def scale(i, _):
    cs = plsc.load_gather(cb, [jnp.full((L,), 0, jnp.int32) + i])
    for j in range(D // L):
      rows[i, pl.ds(j * L, L)] = rows[i, pl.ds(j * L, L)] * cs
    return 0
  lax.fori_loop(0, 128, scale, 0)
  pltpu.sync_copy(rows, agg.at[gid2d.at[0]], add=True)
  sv = srcb[pl.ds(128, L)]
  gv = gidb[pl.ds(128, L)]
  cv = cb[pl.ds(128, L)]
  srcb[pl.ds(0, L)] = sv
  gidb[pl.ds(0, L)] = gv
  cb[pl.ds(0, L)] = cv


def _agg_body(x_hbm, ei2f, cin, aggout, agg, srcv, dstv, cv,
              srcb, gidb, cb, gid2d, rows, zrows, sem):
  c = lax.axis_index("c")
  s = lax.axis_index("s")

  def fz(i, _):
    zrows[i // 8, pl.ds((i % 8) * L, L)] = jnp.zeros((L,), jnp.float32)
    return 0
  lax.fori_loop(0, ZR * D // L, fz, 0)

  def chunk_body(chl, _):
    ch = c * (NCHUNK // NC) + chl
    lo = ch * CH

    def zb(z, _):
      pltpu.sync_copy(zrows, agg.at[pl.ds(_mof(s * ROWS_PT + z * ZR), ZR)])
      return 0
    lax.fori_loop(0, ROWS_PT // ZR, zb, 0)
    plsc.subcore_barrier()

    def r_body(r, _):
      def sub_body(sub, pos):
        eoff = s * EPT + sub * ESUB
        pltpu.sync_copy(ei2f.at[pl.ds(_mof(2 * r * E + eoff), ESUB)], srcv)
        pltpu.sync_copy(ei2f.at[pl.ds(_mof((2 * r + 1) * E + eoff), ESUB)],
                        dstv)
        pltpu.sync_copy(cin.at[pl.ds(_mof(r * E + eoff), ESUB)], cv)

        def step(k, pos):
          d = dstv[pl.ds(k * L, L)]
          dl = d - lo
          m = (dl >= 0) & (dl < CH)
          gi = dl * R + r
          plsc.store_compressed(srcb.at[pl.ds(pos, L)],
                                srcv[pl.ds(k * L, L)], mask=m)
          plsc.store_compressed(gidb.at[pl.ds(pos, L)], gi, mask=m)
          plsc.store_compressed(cb.at[pl.ds(pos, L)], cv[pl.ds(k * L, L)],
                                mask=m)
          pos = pos + jnp.sum(m.astype(jnp.int32))

          def do_flush(p):
            _agg_flush(srcb, gidb, cb, gid2d, rows, sem, x_hbm, agg)
            return p - 128
          return lax.cond(pos >= 128, do_flush, lambda p: p, pos)

        return lax.fori_loop(0, ESUB // L, step, pos)

      pos = lax.fori_loop(0, EPT // ESUB, sub_body, jnp.int32(0))

      i16 = lax.iota(jnp.int32, L)
      for blk in range(128 // L):
        lm = (i16 + blk * L) >= pos
        srcb[pl.ds(blk * L, L)] = jnp.where(lm, 0, srcb[pl.ds(blk * L, L)])
        gidb[pl.ds(blk * L, L)] = jnp.where(lm, 0, gidb[pl.ds(blk * L, L)])
        cb[pl.ds(blk * L, L)] = jnp.where(lm, 0.0, cb[pl.ds(blk * L, L)])
      _agg_flush(srcb, gidb, cb, gid2d, rows, sem, x_hbm, agg)
      return 0

    lax.fori_loop(0, R, r_body, 0)
    plsc.subcore_barrier()
    pltpu.sync_copy(agg.at[pl.ds(_mof(s * ROWS_PT), ROWS_PT)],
                    aggout.at[pl.ds(_mof(ch * AGG_ROWS + s * ROWS_PT),
                                    ROWS_PT)])
    return 0

  lax.fori_loop(0, NCHUNK // NC, chunk_body, 0)
  plsc.subcore_barrier()


def _sc_agg(x, ei2f, cin):
  f = pl.kernel(
      _agg_body,
      out_type=jax.ShapeDtypeStruct((NP * R, D), jnp.float32),
      mesh=plsc.VectorSubcoreMesh(**_MESH),
      compiler_params=pltpu.CompilerParams(needs_layout_passes=False),
      scratch_types=[
          pltpu.VMEM_SHARED((AGG_ROWS, D), jnp.float32),
          pltpu.VMEM((ESUB,), jnp.int32),
          pltpu.VMEM((ESUB,), jnp.int32),
          pltpu.VMEM((ESUB,), jnp.float32),
          pltpu.VMEM((KB,), jnp.int32),
          pltpu.VMEM((KB,), jnp.int32),
          pltpu.VMEM((KB,), jnp.float32),
          pltpu.VMEM((1, 128), jnp.int32),
          pltpu.VMEM((128, D), jnp.float32),
          pltpu.VMEM((ZR, D), jnp.float32),
          pltpu.SemaphoreType.DMA,
      ],
  )
  return f(x, ei2f, cin)


BM = 2048


def _mm_body(a_ref, w_ref, b_ref, o_ref):
  acc = jnp.dot(a_ref[...], w_ref[...],
                preferred_element_type=jnp.float32,
                precision=lax.Precision.HIGHEST)
  o_ref[...] = acc + jnp.mean(b_ref[...], axis=0, keepdims=True)


def _tc_matmul(agg2, wcat, b):
  return pl.pallas_call(
      _mm_body,
      grid=(NP // BM,),
      in_specs=[
          pl.BlockSpec((BM, R * D), lambda i: (i, 0)),
          pl.BlockSpec((R * D, D), lambda i: (0, 0)),
          pl.BlockSpec((R, D), lambda i: (0, 0)),
      ],
      out_specs=pl.BlockSpec((BM, D), lambda i: (i, 0)),
      out_shape=jax.ShapeDtypeStruct((NP, D), jnp.float32),
  )(agg2, wcat, b)


def kernel(node_embedding, edge_index, edge_weight, W, b):
  ei2f = edge_index.astype(jnp.int32).reshape(2 * R * E)
  ewf = edge_weight.astype(jnp.float32).reshape(R * E)
  norms = _sc_degnorm(ei2f)
  cin = _sc_coeff(ei2f, ewf, norms)
  agg = _sc_agg(node_embedding, ei2f, cin)
  agg2 = agg.reshape(NP, R * D)
  out = _tc_matmul(agg2, W.reshape(R * D, D), b)
  return out[:N]
